# Initial kernel scaffold; baseline (speedup 1.0000x reference)
import jax
import jax.numpy as jnp
from jax.experimental import pallas as pl


def kernel(inputs):
    raise NotImplementedError("write your pallas kernel here")


# speedup vs baseline: 4.6060x; 4.6060x over previous
import functools

import jax
import jax.numpy as jnp
from jax import lax
from jax.experimental import pallas as pl
from jax.experimental.pallas import tpu as pltpu
from jax.experimental.pallas import tpu_sc as plsc

R = 64
N = 32768
L = 16
NV = N // L

D0_BITS = 15
D1_BITS = 11
D2_BITS = 6
H0 = 1 << D0_BITS
H1 = 1 << D1_BITS
H2 = 1 << D2_BITS
IDX_MASK = (1 << D0_BITS) - 1


def _srl(x, n):
  return lax.shift_right_logical(x, jnp.int32(n))


def _desc_key(v):
  u = lax.bitcast_convert_type(v, jnp.int32)
  return jnp.where(u >= 0, ~u & jnp.int32(0x7FFFFFFF), u)


def _zero_hist(hist, n, off=0):
  zeros = jnp.zeros((L,), jnp.int32)

  def body(i, _):
    hist[pl.ds(off + i * L, L)] = zeros
    return 0

  lax.fori_loop(0, n // L, body, 0, unroll=4)


def _excl_prefix_sum(hist, n, off=0):
  def body(i, carry):
    v = hist[pl.ds(off + i * L, L)]
    inc = plsc.cumsum(v)
    hist[pl.ds(off + i * L, L)] = inc - v + carry
    return carry + jnp.max(inc)

  lax.fori_loop(0, n // L, body, jnp.int32(0))


def _rr_body(in_hbm, out_hbm, fbuf, bufa, bufb, bufh):
  c = lax.axis_index("c")
  s = lax.axis_index("s")
  wid = s * 2 + c

  for j in range(2):
    row = wid + 32 * j
    pltpu.sync_copy(in_hbm.at[pl.ds(row * N, N)], fbuf)

    _zero_hist(bufb, H0)

    def hist0(i, _):
      k = _desc_key(fbuf[pl.ds(i * L, L)])
      d = k & jnp.int32(H0 - 1)
      cnt, last = plsc.scan_count(d)
      plsc.addupdate_scatter(bufb, [d], cnt, mask=last)
      return 0

    lax.fori_loop(0, NV, hist0, 0)
    _excl_prefix_sum(bufb, H0)

    def scat0(i, _):
      k = _desc_key(fbuf[pl.ds(i * L, L)])
      d = k & jnp.int32(H0 - 1)
      cnt, last = plsc.scan_count(d)
      base = plsc.load_gather(bufb, [d])
      pos = base + cnt - 1
      idx = lax.iota(jnp.int32, L) + i * L
      packed = (k & jnp.int32(~IDX_MASK)) | idx
      plsc.store_scatter(bufa, [pos], packed)
      plsc.addupdate_scatter(bufb, [d], cnt, mask=last)
      return 0

    lax.fori_loop(0, NV, scat0, 0)

    _zero_hist(bufh, H1)

    def hist1(i, _):
      p = bufa[pl.ds(i * L, L)]
      d = _srl(p, D0_BITS) & jnp.int32(H1 - 1)
      cnt, last = plsc.scan_count(d)
      plsc.addupdate_scatter(bufh, [d], cnt, mask=last)
      return 0

    lax.fori_loop(0, NV, hist1, 0)
    _excl_prefix_sum(bufh, H1)

    def scat1(i, _):
      p = bufa[pl.ds(i * L, L)]
      d = _srl(p, D0_BITS) & jnp.int32(H1 - 1)
      cnt, last = plsc.scan_count(d)
      base = plsc.load_gather(bufh, [d])
      pos = base + cnt - 1
      packed = lax.shift_left(_srl(p, D0_BITS + D1_BITS), jnp.int32(D0_BITS)) | (
          p & jnp.int32(IDX_MASK)
      )
      plsc.store_scatter(bufb, [pos], packed)
      plsc.addupdate_scatter(bufh, [d], cnt, mask=last)
      return 0

    lax.fori_loop(0, NV, scat1, 0)

    _zero_hist(bufh, H2)

    def hist2(i, _):
      p = bufb[pl.ds(i * L, L)]
      d = _srl(p, D0_BITS)
      cnt, last = plsc.scan_count(d)
      plsc.addupdate_scatter(bufh, [d], cnt, mask=last)
      return 0

    lax.fori_loop(0, NV, hist2, 0)
    _excl_prefix_sum(bufh, H2)

    def scat2(i, _):
      p = bufb[pl.ds(i * L, L)]
      d = _srl(p, D0_BITS)
      idx = p & jnp.int32(IDX_MASK)
      cnt, last = plsc.scan_count(d)
      base = plsc.load_gather(bufh, [d])
      rank = base + cnt
      w = 1.0 / rank.astype(jnp.float32)
      plsc.store_scatter(fbuf, [idx], w)
      plsc.addupdate_scatter(bufh, [d], cnt, mask=last)
      return 0

    lax.fori_loop(0, NV, scat2, 0)

    pltpu.sync_copy(fbuf, out_hbm.at[pl.ds(row * N, N)])


@jax.jit
def _rr(inputs):
  mesh = plsc.VectorSubcoreMesh(core_axis_name="c", subcore_axis_name="s")
  kfn = pl.kernel(
      _rr_body,
      out_type=jax.ShapeDtypeStruct((R * N,), jnp.float32),
      mesh=mesh,
      compiler_params=pltpu.CompilerParams(needs_layout_passes=False),
      scratch_types=[
          pltpu.VMEM((N,), jnp.float32),
          pltpu.VMEM((N,), jnp.int32),
          pltpu.VMEM((N,), jnp.int32),
          pltpu.VMEM((H1,), jnp.int32),
      ],
  )
  rr = kfn(inputs.reshape(R * N)).reshape(R, N)
  return jnp.where(inputs == 0.0, 0.0, rr)


def kernel(inputs):
  return _rr(inputs)


# speedup vs baseline: 4.6580x; 1.0113x over previous
import functools

import jax
import jax.numpy as jnp
from jax import lax
from jax.experimental import pallas as pl
from jax.experimental.pallas import tpu as pltpu
from jax.experimental.pallas import tpu_sc as plsc

R = 64
N = 32768
L = 16
NV = N // L

D0_BITS = 15
D1_BITS = 11
D2_BITS = 6
H0 = 1 << D0_BITS
H1 = 1 << D1_BITS
H2 = 1 << D2_BITS
IDX_MASK = (1 << D0_BITS) - 1


def _srl(x, n):
  return lax.shift_right_logical(x, jnp.int32(n))


def _desc_key(v):
  u = lax.bitcast_convert_type(v, jnp.int32)
  return jnp.where(u >= 0, ~u & jnp.int32(0x7FFFFFFF), u)


def _zero_hist(hist, n, off=0):
  zeros = jnp.zeros((L,), jnp.int32)

  def body(i, _):
    hist[pl.ds(off + i * L, L)] = zeros
    return 0

  lax.fori_loop(0, n // L, body, 0, unroll=4)


def _excl_prefix_sum(hist, n, off=0):
  def body(i, carry):
    v = hist[pl.ds(off + i * L, L)]
    inc = plsc.cumsum(v)
    hist[pl.ds(off + i * L, L)] = inc - v + carry
    return carry + jnp.max(inc)

  lax.fori_loop(0, n // L, body, jnp.int32(0))


def _rr_body(in_hbm, out_hbm, fbuf, bufa, bufb, bufh):
  c = lax.axis_index("c")
  s = lax.axis_index("s")
  wid = s * 2 + c

  for j in range(2):
    row = wid + 32 * j
    pltpu.sync_copy(in_hbm.at[pl.ds(row * N, N)], fbuf)

    _zero_hist(bufb, H0)

    def hist0(i, _):
      k = _desc_key(fbuf[pl.ds(i * L, L)])
      d = k & jnp.int32(H0 - 1)
      cnt, last = plsc.scan_count(d)
      plsc.addupdate_scatter(bufb, [d], cnt, mask=last)
      return 0

    lax.fori_loop(0, NV, hist0, 0, unroll=4)
    _excl_prefix_sum(bufb, H0)

    def scat0(i, _):
      k = _desc_key(fbuf[pl.ds(i * L, L)])
      d = k & jnp.int32(H0 - 1)
      cnt, last = plsc.scan_count(d)
      base = plsc.load_gather(bufb, [d])
      pos = base + cnt - 1
      idx = lax.iota(jnp.int32, L) + i * L
      packed = (k & jnp.int32(~IDX_MASK)) | idx
      plsc.store_scatter(bufa, [pos], packed)
      plsc.addupdate_scatter(bufb, [d], cnt, mask=last)
      return 0

    lax.fori_loop(0, NV, scat0, 0, unroll=4)

    _zero_hist(bufh, H1)

    def hist1(i, _):
      p = bufa[pl.ds(i * L, L)]
      d = _srl(p, D0_BITS) & jnp.int32(H1 - 1)
      cnt, last = plsc.scan_count(d)
      plsc.addupdate_scatter(bufh, [d], cnt, mask=last)
      return 0

    lax.fori_loop(0, NV, hist1, 0, unroll=4)
    _excl_prefix_sum(bufh, H1)

    def scat1(i, _):
      p = bufa[pl.ds(i * L, L)]
      d = _srl(p, D0_BITS) & jnp.int32(H1 - 1)
      cnt, last = plsc.scan_count(d)
      base = plsc.load_gather(bufh, [d])
      pos = base + cnt - 1
      packed = lax.shift_left(_srl(p, D0_BITS + D1_BITS), jnp.int32(D0_BITS)) | (
          p & jnp.int32(IDX_MASK)
      )
      plsc.store_scatter(bufb, [pos], packed)
      plsc.addupdate_scatter(bufh, [d], cnt, mask=last)
      return 0

    lax.fori_loop(0, NV, scat1, 0, unroll=4)

    _zero_hist(bufh, H2)

    def hist2(i, _):
      p = bufb[pl.ds(i * L, L)]
      d = _srl(p, D0_BITS)
      cnt, last = plsc.scan_count(d)
      plsc.addupdate_scatter(bufh, [d], cnt, mask=last)
      return 0

    lax.fori_loop(0, NV, hist2, 0, unroll=4)
    _excl_prefix_sum(bufh, H2)

    def scat2(i, _):
      p = bufb[pl.ds(i * L, L)]
      d = _srl(p, D0_BITS)
      idx = p & jnp.int32(IDX_MASK)
      cnt, last = plsc.scan_count(d)
      base = plsc.load_gather(bufh, [d])
      rank = base + cnt
      w = 1.0 / rank.astype(jnp.float32)
      plsc.store_scatter(fbuf, [idx], w)
      plsc.addupdate_scatter(bufh, [d], cnt, mask=last)
      return 0

    lax.fori_loop(0, NV, scat2, 0, unroll=4)

    pltpu.sync_copy(fbuf, out_hbm.at[pl.ds(row * N, N)])


@jax.jit
def _rr(inputs):
  mesh = plsc.VectorSubcoreMesh(core_axis_name="c", subcore_axis_name="s")
  kfn = pl.kernel(
      _rr_body,
      out_type=jax.ShapeDtypeStruct((R * N,), jnp.float32),
      mesh=mesh,
      compiler_params=pltpu.CompilerParams(needs_layout_passes=False),
      scratch_types=[
          pltpu.VMEM((N,), jnp.float32),
          pltpu.VMEM((N,), jnp.int32),
          pltpu.VMEM((N,), jnp.int32),
          pltpu.VMEM((H1,), jnp.int32),
      ],
  )
  rr = kfn(inputs.reshape(R * N)).reshape(R, N)
  return jnp.where(inputs == 0.0, 0.0, rr)


def kernel(inputs):
  return _rr(inputs)


# speedup vs baseline: 6.1621x; 1.3229x over previous
import functools

import jax
import jax.numpy as jnp
from jax import lax
from jax.experimental import pallas as pl
from jax.experimental.pallas import tpu as pltpu
from jax.experimental.pallas import tpu_sc as plsc

R = 64
N = 32768
L = 16
NV = N // L

D0_BITS = 15
D1_BITS = 11
D2_BITS = 6
H0 = 1 << D0_BITS
H1 = 1 << D1_BITS
H2 = 1 << D2_BITS
IDX_MASK = (1 << D0_BITS) - 1


def _srl(x, n):
  return lax.shift_right_logical(x, jnp.int32(n))


def _desc_key(v):
  u = lax.bitcast_convert_type(v, jnp.int32)
  return jnp.where(u >= 0, ~u & jnp.int32(0x7FFFFFFF), u)


def _zero_hist(hist, n, off=0):
  zeros = jnp.zeros((L,), jnp.int32)

  def body(i, _):
    hist[pl.ds(off + i * L, L)] = zeros
    return 0

  lax.fori_loop(0, n // L, body, 0, unroll=8)


def _excl_prefix_sum(hist, n, off=0):
  def body(i, carry):
    v = hist[pl.ds(off + i * L, L)]
    inc = plsc.cumsum(v)
    hist[pl.ds(off + i * L, L)] = inc - v + carry
    return carry + jnp.max(inc)

  lax.fori_loop(0, n // L, body, jnp.int32(0))


def _excl_prefix_sum_big(hist, n, sums):
  nv = n // L
  lane_last = lax.iota(jnp.int32, L) == jnp.int32(L - 1)

  def totals(i, _):
    v = hist[pl.ds(i * L, L)]
    inc = plsc.cumsum(v)
    plsc.store_scatter(sums, [jnp.full((L,), i, jnp.int32)], inc, mask=lane_last)
    return 0

  lax.fori_loop(0, nv, totals, 0, unroll=8)

  def scan_tot(i, carry):
    t = sums[pl.ds(i * L, L)]
    inc = plsc.cumsum(t)
    sums[pl.ds(i * L, L)] = inc - t + carry
    return carry + jnp.max(inc)

  lax.fori_loop(0, nv // L, scan_tot, jnp.int32(0))

  def final_group(g, _):
    cvec = sums[pl.ds(g * L, L)]
    for jj in range(L):
      i = g * L + jj
      v = hist[pl.ds(i * L, L)]
      inc = plsc.cumsum(v)
      hist[pl.ds(i * L, L)] = inc - v + cvec[jj]
    return 0

  lax.fori_loop(0, nv // L, final_group, 0)


def _rr_body(in_hbm, out_hbm, fbuf, bufa, bufb, bufh):
  c = lax.axis_index("c")
  s = lax.axis_index("s")
  wid = s * 2 + c

  for j in range(2):
    row = wid + 32 * j
    pltpu.sync_copy(in_hbm.at[pl.ds(row * N, N)], fbuf)

    _zero_hist(bufb, H0)
    _zero_hist(bufh, H1 + H2)

    def hist0(i, _):
      k = _desc_key(fbuf[pl.ds(i * L, L)])
      d = k & jnp.int32(H0 - 1)
      cnt, last = plsc.scan_count(d)
      plsc.addupdate_scatter(bufb, [d], cnt, mask=last)
      return 0

    lax.fori_loop(0, NV, hist0, 0, unroll=4)
    _excl_prefix_sum_big(bufb, H0, bufa)

    def scat0(i, _):
      k = _desc_key(fbuf[pl.ds(i * L, L)])
      d = k & jnp.int32(H0 - 1)
      cnt, last = plsc.scan_count(d)
      base = plsc.load_gather(bufb, [d])
      pos = base + cnt - 1
      idx = lax.iota(jnp.int32, L) + i * L
      packed = (k & jnp.int32(~IDX_MASK)) | idx
      plsc.store_scatter(bufa, [pos], packed)
      plsc.addupdate_scatter(bufb, [d], cnt, mask=last)
      d1 = _srl(k, D0_BITS) & jnp.int32(H1 - 1)
      cnt1, last1 = plsc.scan_count(d1)
      plsc.addupdate_scatter(bufh, [d1], cnt1, mask=last1)
      return 0

    lax.fori_loop(0, NV, scat0, 0, unroll=4)

    _excl_prefix_sum(bufh, H1)

    def scat1(i, _):
      p = bufa[pl.ds(i * L, L)]
      d = _srl(p, D0_BITS) & jnp.int32(H1 - 1)
      cnt, last = plsc.scan_count(d)
      base = plsc.load_gather(bufh, [d])
      pos = base + cnt - 1
      packed = lax.shift_left(_srl(p, D0_BITS + D1_BITS), jnp.int32(D0_BITS)) | (
          p & jnp.int32(IDX_MASK)
      )
      plsc.store_scatter(bufb, [pos], packed)
      plsc.addupdate_scatter(bufh, [d], cnt, mask=last)
      d2 = _srl(p, D0_BITS + D1_BITS) + jnp.int32(H1)
      cnt2, last2 = plsc.scan_count(d2)
      plsc.addupdate_scatter(bufh, [d2], cnt2, mask=last2)
      return 0

    lax.fori_loop(0, NV, scat1, 0, unroll=4)

    _excl_prefix_sum(bufh, H2, off=H1)

    def scat2(i, _):
      p = bufb[pl.ds(i * L, L)]
      d = _srl(p, D0_BITS) + jnp.int32(H1)
      idx = p & jnp.int32(IDX_MASK)
      cnt, last = plsc.scan_count(d)
      base = plsc.load_gather(bufh, [d])
      rank = base + cnt
      w = 1.0 / rank.astype(jnp.float32)
      plsc.store_scatter(fbuf, [idx], w)
      plsc.addupdate_scatter(bufh, [d], cnt, mask=last)
      return 0

    lax.fori_loop(0, NV, scat2, 0, unroll=4)

    pltpu.sync_copy(fbuf, out_hbm.at[pl.ds(row * N, N)])


@jax.jit
def _rr(inputs):
  mesh = plsc.VectorSubcoreMesh(core_axis_name="c", subcore_axis_name="s")
  kfn = pl.kernel(
      _rr_body,
      out_type=jax.ShapeDtypeStruct((R * N,), jnp.float32),
      mesh=mesh,
      compiler_params=pltpu.CompilerParams(needs_layout_passes=False),
      scratch_types=[
          pltpu.VMEM((N,), jnp.float32),
          pltpu.VMEM((N,), jnp.int32),
          pltpu.VMEM((N,), jnp.int32),
          pltpu.VMEM((H1 + H2,), jnp.int32),
      ],
  )
  rr = kfn(inputs.reshape(R * N)).reshape(R, N)
  return jnp.where(inputs == 0.0, 0.0, rr)


def kernel(inputs):
  return _rr(inputs)
